# Initial kernel scaffold; baseline (speedup 1.0000x reference)
import jax
import jax.numpy as jnp
from jax.experimental import pallas as pl


def kernel(z, pos, edge_index, edge_attr, params):
    raise NotImplementedError("write your pallas kernel here")


# speedup vs baseline: 2.0172x; 2.0172x over previous
import functools

import jax
import jax.numpy as jnp
import numpy as np
from jax import lax
from jax.experimental import pallas as pl
from jax.experimental.pallas import tpu as pltpu
from jax.experimental.pallas import tpu_sc as plsc

N = 10000
E = 320000
HID = 128
RBF = 32
CUTOFF = 5.0
GAMMA = 10.0 / (CUTOFF * CUTOFF)

NC, NS = 2, 16
NW = NC * NS
EW = E // NW
K = 80
STEPS = EW // K
CH = 624
TAIL = N - NS * CH

BE = 2000
BN = 2000


def _silu(x):
    return x * jax.nn.sigmoid(x)


def _bdot(a, b):
    return jnp.dot(a.astype(jnp.bfloat16), b.astype(jnp.bfloat16),
                   preferred_element_type=jnp.float32)


def _bf(x):
    bits = lax.bitcast_convert_type(x, jnp.uint32)
    r = bits + jnp.uint32(0x7FFF) + ((bits >> 16) & jnp.uint32(1))
    return lax.bitcast_convert_type(r & jnp.uint32(0xFFFF0000), jnp.float32)


def _ln(x, g, b):
    mu = jnp.mean(x, axis=-1, keepdims=True)
    var = jnp.mean((x - mu) ** 2, axis=-1, keepdims=True)
    return (x - mu) * jax.lax.rsqrt(var + 1e-5) * g + b


def _prep_nodes_body(z_ref, embp_ref, wa_ref, ba_ref, w1a0_ref, h0_ref, hp0_ref):
    zc = jnp.clip(z_ref[...], 0, 100)
    ids = lax.broadcasted_iota(jnp.int32, (BN, HID), 1)
    oh = (ids == zc).astype(jnp.float32)
    he = _bdot(oh, embp_ref[...])
    h0 = _silu(_bdot(he, wa_ref[...]) + ba_ref[...])
    h0_ref[...] = h0
    hp0_ref[...] = _bdot(h0, w1a0_ref[...])


def _prep_edges_body(ea_ref, wep_ref, be_ref, w1b_ref, b1_ref,
                     ep0_ref, ep1_ref, ep2_ref, ep3_ref):
    ea = ea_ref[...]
    dcart = ea[:, 0:3]
    dist = ea[:, 3:4]
    centers = (lax.broadcasted_iota(jnp.int32, (1, RBF), 1).astype(jnp.float32)
               * (CUTOFF / (RBF - 1)))
    rbf = jnp.exp(-GAMMA * (dist - centers) ** 2)
    e_in = jnp.concatenate(
        [rbf, dcart, dist, jnp.zeros((BE, HID - RBF - 4), jnp.float32)], axis=1)
    e = _silu(_bdot(e_in, wep_ref[...]) + be_ref[...])
    t = _bdot(e, w1b_ref[...]) + b1_ref[...]
    ep0_ref[...] = t[:, 0 * HID:1 * HID]
    ep1_ref[...] = t[:, 1 * HID:2 * HID]
    ep2_ref[...] = t[:, 2 * HID:3 * HID]
    ep3_ref[...] = t[:, 3 * HID:4 * HID]


def _edge_body_common(hp_h, ep_h, src_h, dst_h, z128_h, s_out,
                      s_sh, idx_s, idx_d, epb, gb, sem):
    cid = lax.axis_index("c")
    sid = lax.axis_index("s")
    wid = cid * NS + sid
    pltpu.sync_copy(z128_h.at[pl.ds(sid * CH, CH), :],
                    s_sh.at[pl.ds(sid * CH, CH), :])

    @pl.when(sid == NS - 1)
    def _():
        pltpu.sync_copy(z128_h.at[pl.ds(NS * CH, TAIL), :],
                        s_sh.at[pl.ds(NS * CH, TAIL), :])

    plsc.subcore_barrier()
    base = wid * EW

    def step(i, carry):
        off = base + i * K
        pltpu.sync_copy(src_h.at[pl.ds(off, K)], idx_s)
        pltpu.sync_copy(dst_h.at[pl.ds(off, K)], idx_d)
        pltpu.sync_copy(ep_h.at[pl.ds(off, K), :], epb)
        pltpu.async_copy(hp_h.at[idx_s], gb, sem).wait()

        def row(r, c2):
            for j in range(HID // 16):
                sl = pl.ds(j * 16, 16)
                x = gb[r, sl] + epb[r, sl]
                y = x / (1.0 + jnp.exp(-x))
                bits = lax.bitcast_convert_type(y, jnp.uint32)
                rb = bits + jnp.uint32(0x7FFF) + ((bits >> 16) & jnp.uint32(1))
                epb[r, sl] = lax.bitcast_convert_type(
                    rb & jnp.uint32(0xFFFF0000), jnp.float32)
            return c2

        lax.fori_loop(0, K, row, 0)
        pltpu.sync_copy(epb, s_sh.at[idx_d], add=True)
        return carry

    lax.fori_loop(0, STEPS, step, 0)
    plsc.subcore_barrier()
    pltpu.sync_copy(s_sh.at[pl.ds(sid * CH, CH), :],
                    s_out.at[cid, pl.ds(sid * CH, CH), :])

    @pl.when(sid == NS - 1)
    def _():
        pltpu.sync_copy(s_sh.at[pl.ds(NS * CH, TAIL), :],
                        s_out.at[cid, pl.ds(NS * CH, TAIL), :])


def _deg_body(dst_h, z128_h, deg_out, s_sh, idx_d, ones_b):
    cid = lax.axis_index("c")
    sid = lax.axis_index("s")
    wid = cid * NS + sid
    pltpu.sync_copy(z128_h.at[pl.ds(sid * CH, CH), :],
                    s_sh.at[pl.ds(sid * CH, CH), :])

    @pl.when(sid == NS - 1)
    def _():
        pltpu.sync_copy(z128_h.at[pl.ds(NS * CH, TAIL), :],
                        s_sh.at[pl.ds(NS * CH, TAIL), :])

    def fill(r, c2):
        for j in range(HID // 16):
            ones_b[r, pl.ds(j * 16, 16)] = jnp.full((16,), 1.0, jnp.float32)
        return c2

    lax.fori_loop(0, K, fill, 0)
    plsc.subcore_barrier()
    base = wid * EW

    def step(i, carry):
        off = base + i * K
        pltpu.sync_copy(dst_h.at[pl.ds(off, K)], idx_d)
        pltpu.sync_copy(ones_b, s_sh.at[idx_d], add=True)
        return carry

    lax.fori_loop(0, STEPS, step, 0)
    plsc.subcore_barrier()
    pltpu.sync_copy(s_sh.at[pl.ds(sid * CH, CH), :],
                    deg_out.at[cid, pl.ds(sid * CH, CH), :])

    @pl.when(sid == NS - 1)
    def _():
        pltpu.sync_copy(s_sh.at[pl.ds(NS * CH, TAIL), :],
                        deg_out.at[cid, pl.ds(NS * CH, TAIL), :])


_SC_MESH = plsc.VectorSubcoreMesh(core_axis_name="c", subcore_axis_name="s",
                                  num_cores=NC, num_subcores=NS)


def _edge_sc(hp, ep, src, dst, z128):
    fn = pl.kernel(
        _edge_body_common,
        out_type=jax.ShapeDtypeStruct((NC, N, HID), jnp.float32),
        mesh=_SC_MESH,
        scratch_types=[
            pltpu.VMEM_SHARED((N, HID), jnp.float32),
            pltpu.VMEM((K,), jnp.int32),
            pltpu.VMEM((K,), jnp.int32),
            pltpu.VMEM((K, HID), jnp.float32),
            pltpu.VMEM((K, HID), jnp.float32),
            pltpu.SemaphoreType.DMA,
        ],
    )
    return fn(hp, ep, src, dst, z128)


def _deg_sc(dst, z128):
    fn = pl.kernel(
        _deg_body,
        out_type=jax.ShapeDtypeStruct((NC, N, HID), jnp.float32),
        mesh=_SC_MESH,
        scratch_types=[
            pltpu.VMEM_SHARED((N, HID), jnp.float32),
            pltpu.VMEM((K,), jnp.int32),
            pltpu.VMEM((K, HID), jnp.float32),
        ],
    )
    return fn(dst, z128)


def _node_body(s2_ref, deg2_ref, h_ref, w2_ref, b2_ref, g1_ref, bn1_ref,
               wu1_ref, bu1_ref, wu2_ref, bu2_ref, g2_ref, bn2_ref,
               w1an_ref, h_out_ref, hp_out_ref):
    s = s2_ref[0] + s2_ref[1]
    deg = (deg2_ref[0] + deg2_ref[1])[:, 0:1]
    s_hi = _bf(s)
    r = s - s_hi
    r_hi = _bf(r)
    r2_hi = _bf(r - r_hi)
    agg = (_bdot(s_hi, w2_ref[...]) + _bdot(r_hi, w2_ref[...])
           + _bdot(r2_hi, w2_ref[...]) + deg * b2_ref[...])
    h1 = _ln(h_ref[...] + agg, g1_ref[...], bn1_ref[...])
    cat = jnp.concatenate([h1, agg], axis=1)
    u = _silu(_bdot(cat, wu1_ref[...]) + bu1_ref[...])
    u = _bdot(u, wu2_ref[...]) + bu2_ref[...]
    h2 = _ln(h1 + u, g2_ref[...], bn2_ref[...])
    h_out_ref[...] = h2
    hp_out_ref[...] = _bdot(h2, w1an_ref[...])


def _node_final_body(s2_ref, deg2_ref, h_ref, w2_ref, b2_ref, g1_ref, bn1_ref,
                     wu1_ref, bu1_ref, wu2_ref, bu2_ref, g2_ref, bn2_ref,
                     wd1_ref, bd1_ref, wd2_ref, bd2_ref,
                     wr1_ref, br1_ref, wr2_ref, br2_ref, dr_ref, rm_ref):
    s = s2_ref[0] + s2_ref[1]
    deg = (deg2_ref[0] + deg2_ref[1])[:, 0:1]
    s_hi = _bf(s)
    r = s - s_hi
    r_hi = _bf(r)
    r2_hi = _bf(r - r_hi)
    agg = (_bdot(s_hi, w2_ref[...]) + _bdot(r_hi, w2_ref[...])
           + _bdot(r2_hi, w2_ref[...]) + deg * b2_ref[...])
    h1 = _ln(h_ref[...] + agg, g1_ref[...], bn1_ref[...])
    cat = jnp.concatenate([h1, agg], axis=1)
    u = _silu(_bdot(cat, wu1_ref[...]) + bu1_ref[...])
    u = _bdot(u, wu2_ref[...]) + bu2_ref[...]
    h2 = _ln(h1 + u, g2_ref[...], bn2_ref[...])
    d = _silu(_bdot(h2, wd1_ref[...]) + bd1_ref[...])
    dr_ref[...] = _bdot(d, wd2_ref[...]) + bd2_ref[...]
    r = _silu(_bdot(h2, wr1_ref[...]) + br1_ref[...])
    rm_ref[...] = _bdot(r, wr2_ref[...]) + br2_ref[...]


def _full(shape):
    return pl.BlockSpec(shape, lambda i: tuple(0 for _ in shape))


def _rows(shape):
    nd = len(shape)
    if nd == 2:
        return pl.BlockSpec(shape, lambda i: (i, 0))
    return pl.BlockSpec(shape, lambda i: (0, i, 0))


def kernel(z, pos, edge_index, edge_attr, params):
    p = params
    emb = p['emb']
    wa, ba = p['atom_lin']
    we, be_ = p['edge_lin']
    mp = p['mp']

    embp = jnp.zeros((HID, HID), jnp.float32).at[:emb.shape[0]].set(emb)
    wep = jnp.zeros((HID, HID), jnp.float32).at[:we.shape[0]].set(we)
    w1a = [mp[l]['msg1'][0][:HID] for l in range(4)]
    w1ball = jnp.concatenate([mp[l]['msg1'][0][HID:] for l in range(4)], axis=1)
    b1all = jnp.concatenate([mp[l]['msg1'][1] for l in range(4)])[None, :]

    z2 = z.reshape(N, 1).astype(jnp.int32)
    src = edge_index[0].astype(jnp.int32)
    dst = edge_index[1].astype(jnp.int32)
    z128 = jnp.zeros((N, HID), jnp.float32)

    h0, hp = pl.pallas_call(
        _prep_nodes_body,
        grid=(N // BN,),
        in_specs=[pl.BlockSpec((BN, 1), lambda i: (i, 0)),
                  _full((HID, HID)), _full((HID, HID)), _full((1, HID)),
                  _full((HID, HID))],
        out_specs=[_rows((BN, HID)), _rows((BN, HID))],
        out_shape=[jax.ShapeDtypeStruct((N, HID), jnp.float32),
                   jax.ShapeDtypeStruct((N, HID), jnp.float32)],
    )(z2, embp, wa, ba[None, :], w1a[0])

    eps = pl.pallas_call(
        _prep_edges_body,
        grid=(E // BE,),
        in_specs=[pl.BlockSpec((BE, 4), lambda i: (i, 0)),
                  _full((HID, HID)), _full((1, HID)),
                  _full((HID, 4 * HID)), _full((1, 4 * HID))],
        out_specs=[_rows((BE, HID))] * 4,
        out_shape=[jax.ShapeDtypeStruct((E, HID), jnp.float32)] * 4,
    )(edge_attr, wep, be_[None, :], w1ball, b1all)

    h = h0
    deg2 = _deg_sc(dst, z128)
    dr = rm = None
    for l in range(4):
        layer = mp[l]
        s2 = _edge_sc(hp, eps[l], src, dst, z128)
        w2, b2 = layer['msg2']
        g1, bn1 = layer['n1']
        wu1, bu1 = layer['upd1']
        wu2, bu2 = layer['upd2']
        g2, bn2 = layer['n2']
        if l < 3:
            h, hp = pl.pallas_call(
                _node_body,
                grid=(N // BN,),
                in_specs=[_rows((NC, BN, HID)), _rows((NC, BN, HID)),
                          _rows((BN, HID)),
                          _full((HID, HID)), _full((1, HID)),
                          _full((1, HID)), _full((1, HID)),
                          _full((2 * HID, HID)), _full((1, HID)),
                          _full((HID, HID)), _full((1, HID)),
                          _full((1, HID)), _full((1, HID)),
                          _full((HID, HID))],
                out_specs=[_rows((BN, HID)), _rows((BN, HID))],
                out_shape=[jax.ShapeDtypeStruct((N, HID), jnp.float32),
                           jax.ShapeDtypeStruct((N, HID), jnp.float32)],
            )(s2, deg2, h, w2, b2[None, :], g1[None, :], bn1[None, :],
              wu1, bu1[None, :], wu2, bu2[None, :], g2[None, :], bn2[None, :],
              w1a[l + 1])
        else:
            wd1, bd1 = p['head_dr1']
            wd2, bd2 = p['head_dr2']
            wr1, br1 = p['head_rm1']
            wr2, br2 = p['head_rm2']
            dr, rm = pl.pallas_call(
                _node_final_body,
                grid=(N // BN,),
                in_specs=[_rows((NC, BN, HID)), _rows((NC, BN, HID)),
                          _rows((BN, HID)),
                          _full((HID, HID)), _full((1, HID)),
                          _full((1, HID)), _full((1, HID)),
                          _full((2 * HID, HID)), _full((1, HID)),
                          _full((HID, HID)), _full((1, HID)),
                          _full((1, HID)), _full((1, HID)),
                          _full((HID, HID)), _full((1, HID)),
                          _full((HID, 3)), _full((1, 3)),
                          _full((HID, HID)), _full((1, HID)),
                          _full((HID, 1)), _full((1, 1))],
                out_specs=[_rows((BN, 3)), _rows((BN, 1))],
                out_shape=[jax.ShapeDtypeStruct((N, 3), jnp.float32),
                           jax.ShapeDtypeStruct((N, 1), jnp.float32)],
            )(s2, deg2, h, w2, b2[None, :], g1[None, :], bn1[None, :],
              wu1, bu1[None, :], wu2, bu2[None, :], g2[None, :], bn2[None, :],
              wd1, bd1[None, :], wd2, bd2[None, :],
              wr1, br1[None, :], wr2, br2[None, :])
    return (dr, rm)


# speedup vs baseline: 3.0207x; 1.4974x over previous
import functools

import jax
import jax.numpy as jnp
import numpy as np
from jax import lax
from jax.experimental import pallas as pl
from jax.experimental.pallas import tpu as pltpu
from jax.experimental.pallas import tpu_sc as plsc

N = 10000
E = 320000
HID = 128
RBF = 32
CUTOFF = 5.0
GAMMA = 10.0 / (CUTOFF * CUTOFF)

NC, NS = 2, 16
NW = NC * NS
EW = E // NW
K = 80
STEPS = EW // K
CH = 624
TAIL = N - NS * CH

BE = 2000
BN = 2000


def _silu(x):
    return x * jax.nn.sigmoid(x)


def _bdot(a, b):
    return jnp.dot(a.astype(jnp.bfloat16), b.astype(jnp.bfloat16),
                   preferred_element_type=jnp.float32)


def _bf(x):
    bits = lax.bitcast_convert_type(x, jnp.uint32)
    r = bits + jnp.uint32(0x7FFF) + ((bits >> 16) & jnp.uint32(1))
    return lax.bitcast_convert_type(r & jnp.uint32(0xFFFF0000), jnp.float32)


def _ln(x, g, b):
    mu = jnp.mean(x, axis=-1, keepdims=True)
    var = jnp.mean((x - mu) ** 2, axis=-1, keepdims=True)
    return (x - mu) * jax.lax.rsqrt(var + 1e-5) * g + b


def _prep_nodes_body(z_ref, embp_ref, wa_ref, ba_ref, w1a0_ref, h0_ref, hp0_ref):
    zc = jnp.clip(z_ref[...], 0, 100)
    ids = lax.broadcasted_iota(jnp.int32, (BN, HID), 1)
    oh = (ids == zc).astype(jnp.float32)
    he = _bdot(oh, embp_ref[...])
    h0 = _silu(_bdot(he, wa_ref[...]) + ba_ref[...])
    h0_ref[...] = h0
    hp0_ref[...] = _bdot(h0, w1a0_ref[...])


def _prep_edges_body(ea_ref, wep_ref, be_ref, w1b_ref, b1_ref,
                     ep0_ref, ep1_ref, ep2_ref, ep3_ref):
    ea = ea_ref[...]
    dcart = ea[:, 0:3]
    dist = ea[:, 3:4]
    centers = (lax.broadcasted_iota(jnp.int32, (1, RBF), 1).astype(jnp.float32)
               * (CUTOFF / (RBF - 1)))
    rbf = jnp.exp(-GAMMA * (dist - centers) ** 2)
    e_in = jnp.concatenate(
        [rbf, dcart, dist, jnp.zeros((BE, HID - RBF - 4), jnp.float32)], axis=1)
    e = _silu(_bdot(e_in, wep_ref[...]) + be_ref[...])
    t = _bdot(e, w1b_ref[...]) + b1_ref[...]
    ep0_ref[...] = t[:, 0 * HID:1 * HID]
    ep1_ref[...] = t[:, 1 * HID:2 * HID]
    ep2_ref[...] = t[:, 2 * HID:3 * HID]
    ep3_ref[...] = t[:, 3 * HID:4 * HID]


def _edge_compute_block(gb, epb):
    def row(r, c2):
        for j in range(HID // 16):
            sl = pl.ds(j * 16, 16)
            x = gb[r, sl] + epb[r, sl]
            y = x / (1.0 + jnp.exp(-x))
            bits = lax.bitcast_convert_type(y, jnp.uint32)
            rb = bits + jnp.uint32(0x7FFF) + ((bits >> 16) & jnp.uint32(1))
            epb[r, sl] = lax.bitcast_convert_type(
                rb & jnp.uint32(0xFFFF0000), jnp.float32)
        return c2

    lax.fori_loop(0, K, row, 0)


def _edge_body_common(hp_h, ep_h, src_h, dst_h, z128_h, s_out,
                      s_sh, idx_s0, idx_d0, epb0, gb0,
                      idx_s1, idx_d1, epb1, gb1,
                      sem_l0, sem_l1, sem_g0, sem_g1):
    cid = lax.axis_index("c")
    sid = lax.axis_index("s")
    wid = cid * NS + sid
    pltpu.sync_copy(z128_h.at[pl.ds(sid * CH, CH), :],
                    s_sh.at[pl.ds(sid * CH, CH), :])

    @pl.when(sid == NS - 1)
    def _():
        pltpu.sync_copy(z128_h.at[pl.ds(NS * CH, TAIL), :],
                        s_sh.at[pl.ds(NS * CH, TAIL), :])

    plsc.subcore_barrier()
    base = wid * EW
    slots = ((idx_s0, idx_d0, epb0, gb0, sem_l0, sem_g0),
             (idx_s1, idx_d1, epb1, gb1, sem_l1, sem_g1))

    def load(off, sl):
        d1 = pltpu.async_copy(src_h.at[pl.ds(off, K)], sl[0], sl[4])
        d2 = pltpu.async_copy(dst_h.at[pl.ds(off, K)], sl[1], sl[4])
        d3 = pltpu.async_copy(ep_h.at[pl.ds(off, K), :], sl[2], sl[4])
        return d1, d2, d3

    def drain_gather(sl):
        pltpu.make_async_copy(z128_h.at[pl.ds(0, K), :], sl[3], sl[5]).wait()

    for d in load(base, slots[0]):
        d.wait()
    pltpu.async_copy(hp_h.at[slots[0][0]], slots[0][3], slots[0][5])

    def pair(g, carry):
        for b in range(2):
            i = 2 * g + b
            cur = slots[b]
            nxt = slots[1 - b]
            nld = load(base + (i + 1) * K, nxt)
            drain_gather(cur)
            _edge_compute_block(cur[3], cur[2])
            for d in nld:
                d.wait()
            pltpu.async_copy(hp_h.at[nxt[0]], nxt[3], nxt[5])
            pltpu.sync_copy(cur[2], s_sh.at[cur[1]], add=True)
        return carry

    lax.fori_loop(0, (STEPS - 1) // 2, pair, 0)
    cur = slots[(STEPS - 1) % 2]
    drain_gather(cur)
    _edge_compute_block(cur[3], cur[2])
    pltpu.sync_copy(cur[2], s_sh.at[cur[1]], add=True)

    plsc.subcore_barrier()
    pltpu.sync_copy(s_sh.at[pl.ds(sid * CH, CH), :],
                    s_out.at[cid, pl.ds(sid * CH, CH), :])

    @pl.when(sid == NS - 1)
    def _():
        pltpu.sync_copy(s_sh.at[pl.ds(NS * CH, TAIL), :],
                        s_out.at[cid, pl.ds(NS * CH, TAIL), :])


def _deg_body(dst_h, z128_h, deg_out, s_sh, idx_d, ones_b):
    cid = lax.axis_index("c")
    sid = lax.axis_index("s")
    wid = cid * NS + sid
    pltpu.sync_copy(z128_h.at[pl.ds(sid * CH, CH), :],
                    s_sh.at[pl.ds(sid * CH, CH), :])

    @pl.when(sid == NS - 1)
    def _():
        pltpu.sync_copy(z128_h.at[pl.ds(NS * CH, TAIL), :],
                        s_sh.at[pl.ds(NS * CH, TAIL), :])

    def fill(r, c2):
        for j in range(HID // 16):
            ones_b[r, pl.ds(j * 16, 16)] = jnp.full((16,), 1.0, jnp.float32)
        return c2

    lax.fori_loop(0, K, fill, 0)
    plsc.subcore_barrier()
    base = wid * EW

    def step(i, carry):
        off = base + i * K
        pltpu.sync_copy(dst_h.at[pl.ds(off, K)], idx_d)
        pltpu.sync_copy(ones_b, s_sh.at[idx_d], add=True)
        return carry

    lax.fori_loop(0, STEPS, step, 0)
    plsc.subcore_barrier()
    pltpu.sync_copy(s_sh.at[pl.ds(sid * CH, CH), :],
                    deg_out.at[cid, pl.ds(sid * CH, CH), :])

    @pl.when(sid == NS - 1)
    def _():
        pltpu.sync_copy(s_sh.at[pl.ds(NS * CH, TAIL), :],
                        deg_out.at[cid, pl.ds(NS * CH, TAIL), :])


_SC_MESH = plsc.VectorSubcoreMesh(core_axis_name="c", subcore_axis_name="s",
                                  num_cores=NC, num_subcores=NS)


def _edge_sc(hp, ep, src, dst, z128):
    fn = pl.kernel(
        _edge_body_common,
        out_type=jax.ShapeDtypeStruct((NC, N, HID), jnp.float32),
        mesh=_SC_MESH,
        scratch_types=[
            pltpu.VMEM_SHARED((N, HID), jnp.float32),
            pltpu.VMEM((K,), jnp.int32),
            pltpu.VMEM((K,), jnp.int32),
            pltpu.VMEM((K, HID), jnp.float32),
            pltpu.VMEM((K, HID), jnp.float32),
            pltpu.VMEM((K,), jnp.int32),
            pltpu.VMEM((K,), jnp.int32),
            pltpu.VMEM((K, HID), jnp.float32),
            pltpu.VMEM((K, HID), jnp.float32),
            pltpu.SemaphoreType.DMA,
            pltpu.SemaphoreType.DMA,
            pltpu.SemaphoreType.DMA,
            pltpu.SemaphoreType.DMA,
        ],
    )
    return fn(hp, ep, src, dst, z128)


def _deg_sc(dst, z128):
    fn = pl.kernel(
        _deg_body,
        out_type=jax.ShapeDtypeStruct((NC, N, HID), jnp.float32),
        mesh=_SC_MESH,
        scratch_types=[
            pltpu.VMEM_SHARED((N, HID), jnp.float32),
            pltpu.VMEM((K,), jnp.int32),
            pltpu.VMEM((K, HID), jnp.float32),
        ],
    )
    return fn(dst, z128)


def _node_body(s2_ref, deg2_ref, h_ref, w2_ref, b2_ref, g1_ref, bn1_ref,
               wu1_ref, bu1_ref, wu2_ref, bu2_ref, g2_ref, bn2_ref,
               w1an_ref, h_out_ref, hp_out_ref):
    s = s2_ref[0] + s2_ref[1]
    deg = (deg2_ref[0] + deg2_ref[1])[:, 0:1]
    s_hi = _bf(s)
    r = s - s_hi
    r_hi = _bf(r)
    r2_hi = _bf(r - r_hi)
    agg = (_bdot(s_hi, w2_ref[...]) + _bdot(r_hi, w2_ref[...])
           + _bdot(r2_hi, w2_ref[...]) + deg * b2_ref[...])
    h1 = _ln(h_ref[...] + agg, g1_ref[...], bn1_ref[...])
    cat = jnp.concatenate([h1, agg], axis=1)
    u = _silu(_bdot(cat, wu1_ref[...]) + bu1_ref[...])
    u = _bdot(u, wu2_ref[...]) + bu2_ref[...]
    h2 = _ln(h1 + u, g2_ref[...], bn2_ref[...])
    h_out_ref[...] = h2
    hp_out_ref[...] = _bdot(h2, w1an_ref[...])


def _node_final_body(s2_ref, deg2_ref, h_ref, w2_ref, b2_ref, g1_ref, bn1_ref,
                     wu1_ref, bu1_ref, wu2_ref, bu2_ref, g2_ref, bn2_ref,
                     wd1_ref, bd1_ref, wd2_ref, bd2_ref,
                     wr1_ref, br1_ref, wr2_ref, br2_ref, dr_ref, rm_ref):
    s = s2_ref[0] + s2_ref[1]
    deg = (deg2_ref[0] + deg2_ref[1])[:, 0:1]
    s_hi = _bf(s)
    r = s - s_hi
    r_hi = _bf(r)
    r2_hi = _bf(r - r_hi)
    agg = (_bdot(s_hi, w2_ref[...]) + _bdot(r_hi, w2_ref[...])
           + _bdot(r2_hi, w2_ref[...]) + deg * b2_ref[...])
    h1 = _ln(h_ref[...] + agg, g1_ref[...], bn1_ref[...])
    cat = jnp.concatenate([h1, agg], axis=1)
    u = _silu(_bdot(cat, wu1_ref[...]) + bu1_ref[...])
    u = _bdot(u, wu2_ref[...]) + bu2_ref[...]
    h2 = _ln(h1 + u, g2_ref[...], bn2_ref[...])
    d = _silu(_bdot(h2, wd1_ref[...]) + bd1_ref[...])
    dr_ref[...] = _bdot(d, wd2_ref[...]) + bd2_ref[...]
    r = _silu(_bdot(h2, wr1_ref[...]) + br1_ref[...])
    rm_ref[...] = _bdot(r, wr2_ref[...]) + br2_ref[...]


def _full(shape):
    return pl.BlockSpec(shape, lambda i: tuple(0 for _ in shape))


def _rows(shape):
    nd = len(shape)
    if nd == 2:
        return pl.BlockSpec(shape, lambda i: (i, 0))
    return pl.BlockSpec(shape, lambda i: (0, i, 0))


def kernel(z, pos, edge_index, edge_attr, params):
    p = params
    emb = p['emb']
    wa, ba = p['atom_lin']
    we, be_ = p['edge_lin']
    mp = p['mp']

    embp = jnp.zeros((HID, HID), jnp.float32).at[:emb.shape[0]].set(emb)
    wep = jnp.zeros((HID, HID), jnp.float32).at[:we.shape[0]].set(we)
    w1a = [mp[l]['msg1'][0][:HID] for l in range(4)]
    w1ball = jnp.concatenate([mp[l]['msg1'][0][HID:] for l in range(4)], axis=1)
    b1all = jnp.concatenate([mp[l]['msg1'][1] for l in range(4)])[None, :]

    z2 = z.reshape(N, 1).astype(jnp.int32)
    src = edge_index[0].astype(jnp.int32)
    dst = edge_index[1].astype(jnp.int32)
    z128 = jnp.zeros((N, HID), jnp.float32)

    h0, hp = pl.pallas_call(
        _prep_nodes_body,
        grid=(N // BN,),
        in_specs=[pl.BlockSpec((BN, 1), lambda i: (i, 0)),
                  _full((HID, HID)), _full((HID, HID)), _full((1, HID)),
                  _full((HID, HID))],
        out_specs=[_rows((BN, HID)), _rows((BN, HID))],
        out_shape=[jax.ShapeDtypeStruct((N, HID), jnp.float32),
                   jax.ShapeDtypeStruct((N, HID), jnp.float32)],
    )(z2, embp, wa, ba[None, :], w1a[0])

    eps = pl.pallas_call(
        _prep_edges_body,
        grid=(E // BE,),
        in_specs=[pl.BlockSpec((BE, 4), lambda i: (i, 0)),
                  _full((HID, HID)), _full((1, HID)),
                  _full((HID, 4 * HID)), _full((1, 4 * HID))],
        out_specs=[_rows((BE, HID))] * 4,
        out_shape=[jax.ShapeDtypeStruct((E, HID), jnp.float32)] * 4,
    )(edge_attr, wep, be_[None, :], w1ball, b1all)

    h = h0
    deg2 = _deg_sc(dst, z128)
    dr = rm = None
    for l in range(4):
        layer = mp[l]
        s2 = _edge_sc(hp, eps[l], src, dst, z128)
        w2, b2 = layer['msg2']
        g1, bn1 = layer['n1']
        wu1, bu1 = layer['upd1']
        wu2, bu2 = layer['upd2']
        g2, bn2 = layer['n2']
        if l < 3:
            h, hp = pl.pallas_call(
                _node_body,
                grid=(N // BN,),
                in_specs=[_rows((NC, BN, HID)), _rows((NC, BN, HID)),
                          _rows((BN, HID)),
                          _full((HID, HID)), _full((1, HID)),
                          _full((1, HID)), _full((1, HID)),
                          _full((2 * HID, HID)), _full((1, HID)),
                          _full((HID, HID)), _full((1, HID)),
                          _full((1, HID)), _full((1, HID)),
                          _full((HID, HID))],
                out_specs=[_rows((BN, HID)), _rows((BN, HID))],
                out_shape=[jax.ShapeDtypeStruct((N, HID), jnp.float32),
                           jax.ShapeDtypeStruct((N, HID), jnp.float32)],
            )(s2, deg2, h, w2, b2[None, :], g1[None, :], bn1[None, :],
              wu1, bu1[None, :], wu2, bu2[None, :], g2[None, :], bn2[None, :],
              w1a[l + 1])
        else:
            wd1, bd1 = p['head_dr1']
            wd2, bd2 = p['head_dr2']
            wr1, br1 = p['head_rm1']
            wr2, br2 = p['head_rm2']
            dr, rm = pl.pallas_call(
                _node_final_body,
                grid=(N // BN,),
                in_specs=[_rows((NC, BN, HID)), _rows((NC, BN, HID)),
                          _rows((BN, HID)),
                          _full((HID, HID)), _full((1, HID)),
                          _full((1, HID)), _full((1, HID)),
                          _full((2 * HID, HID)), _full((1, HID)),
                          _full((HID, HID)), _full((1, HID)),
                          _full((1, HID)), _full((1, HID)),
                          _full((HID, HID)), _full((1, HID)),
                          _full((HID, 3)), _full((1, 3)),
                          _full((HID, HID)), _full((1, HID)),
                          _full((HID, 1)), _full((1, 1))],
                out_specs=[_rows((BN, 3)), _rows((BN, 1))],
                out_shape=[jax.ShapeDtypeStruct((N, 3), jnp.float32),
                           jax.ShapeDtypeStruct((N, 1), jnp.float32)],
            )(s2, deg2, h, w2, b2[None, :], g1[None, :], bn1[None, :],
              wu1, bu1[None, :], wu2, bu2[None, :], g2[None, :], bn2[None, :],
              wd1, bd1[None, :], wd2, bd2[None, :],
              wr1, br1[None, :], wr2, br2[None, :])
    return (dr, rm)
